# baseline (speedup 1.0000x reference)
import functools

import jax
import jax.numpy as jnp
from jax import lax
from jax.experimental import pallas as pl
from jax.experimental.pallas import tpu as pltpu
from jax.experimental.pallas import tpu_sc as plsc

N_NODES = 10000
N_EDGES = 320000
CH = 128
EA = 16

NC = 2
NS = 16
L = 16
NW = NC * NS
EPW = N_EDGES // NW
STEPS = EPW // L
ZSTEPS = N_NODES // L
E128 = N_EDGES * EA // CH

_HI = lax.Precision.HIGHEST


def _prep_body(x_ref, gW_ref, gb_ref, fW_ref, fb_ref, cW_ref, cb_ref,
               node_ref, c_ref):
    cW = cW_ref[...]
    fW = fW_ref[...]
    u = lax.dot_general(fW[0:CH], cW, (((1,), (0,)), ((), ())), precision=_HI)
    v = lax.dot_general(fW[CH:2 * CH], cW, (((1,), (0,)), ((), ())), precision=_HI)
    gW = gW_ref[...]
    a = lax.dot_general(gW[0:CH], v, (((1,), (0,)), ((), ())), precision=_HI)
    b = lax.dot_general(gW[CH:2 * CH], v, (((1,), (0,)), ((), ())), precision=_HI)
    c = lax.dot_general(gW[2 * CH:2 * CH + EA], v, (((1,), (0,)), ((), ())), precision=_HI)
    s0 = jnp.sum(gb_ref[...] * v[:, 0])
    s1 = jnp.sum(fb_ref[...] * cW[:, 0]) + jnp.sum(cb_ref[...])
    W3 = jnp.concatenate([u, a, b, jnp.zeros((CH, 5), jnp.float32)], axis=1)
    node = lax.dot_general(W3, x_ref[...], (((0,), (1,)), ((), ())), precision=_HI)
    row = lax.broadcasted_iota(jnp.int32, node.shape, 0)
    node_ref[...] = node + jnp.where(row == 0, s1, 0.0) + jnp.where(row == 1, s0, 0.0)
    c_ref[...] = c


def _edge_body(X_ref, ei_ref, c_ref, r_ref, src_ref, dst_ref):
    acc = X_ref[0:1, :] * c_ref[0:1, 0:1]
    for j in range(1, EA):
        acc = acc + X_ref[j:j + 1, :] * c_ref[j:j + 1, 0:1]
    r_ref[...] = acc
    src_ref[...] = ei_ref[0:1, :]
    dst_ref[...] = ei_ref[1:2, :]


def _combine_body(node_ref, T_ref, cnt_ref, out_ref):
    T = jnp.sum(T_ref[...], axis=0, keepdims=True)
    cnt = jnp.sum(cnt_ref[...], axis=0, keepdims=True)
    xu = node_ref[0:1, :]
    xa = node_ref[1:2, :]
    out_ref[...] = xu + jnp.where(cnt > 0.0, xa + T / jnp.maximum(cnt, 1.0), 0.0)


@functools.cache
def _sc_segsum_kernel():
    return pl.kernel(
        _sc_segsum_body,
        out_type=[jax.ShapeDtypeStruct((NW, N_NODES), jnp.float32),
                  jax.ShapeDtypeStruct((NW, N_NODES), jnp.float32)],
        mesh=plsc.VectorSubcoreMesh(core_axis_name="c", subcore_axis_name="s",
                                    num_cores=NC, num_subcores=NS),
        compiler_params=pltpu.CompilerParams(needs_layout_passes=False),
        scratch_types=[pltpu.VMEM((EPW,), jnp.int32),
                       pltpu.VMEM((EPW,), jnp.int32),
                       pltpu.VMEM((EPW,), jnp.float32),
                       pltpu.VMEM((N_NODES,), jnp.float32),
                       pltpu.VMEM((N_NODES,), jnp.float32),
                       pltpu.VMEM((N_NODES,), jnp.float32)],
    )


def _sc_segsum_body(src_hbm, dst_hbm, r_hbm, p_hbm, T_hbm, cnt_hbm,
                    src_v, dst_v, r_v, p_v, T_v, cnt_v):
    wid = lax.axis_index("s") * NC + lax.axis_index("c")
    base = wid * EPW
    pltpu.sync_copy(src_hbm.at[pl.ds(base, EPW)], src_v)
    pltpu.sync_copy(dst_hbm.at[pl.ds(base, EPW)], dst_v)
    pltpu.sync_copy(r_hbm.at[pl.ds(base, EPW)], r_v)
    pltpu.sync_copy(p_hbm, p_v)

    def zero_body(i, carry):
        off = i * L
        z = jnp.zeros((L,), jnp.float32)
        T_v[pl.ds(off, L)] = z
        cnt_v[pl.ds(off, L)] = z
        return carry

    lax.fori_loop(0, ZSTEPS, zero_body, 0)

    def body(i, carry):
        off = i * L
        s = src_v[pl.ds(off, L)]
        d = dst_v[pl.ds(off, L)]
        rv = r_v[pl.ds(off, L)]
        pv = plsc.load_gather(p_v, [s])
        plsc.addupdate_scatter(T_v, [d], pv + rv)
        plsc.addupdate_scatter(cnt_v, [d], jnp.full((L,), 1.0, jnp.float32))
        return carry

    lax.fori_loop(0, STEPS, body, 0)

    pltpu.sync_copy(T_v, T_hbm.at[wid])
    pltpu.sync_copy(cnt_v, cnt_hbm.at[wid])


def _segment_parts(src, dst, r, p):
    return _sc_segsum_kernel()(src, dst, r, p)


def kernel(x_a, edge_index_a2a, edge_attr_a2a, g_W, g_b, f_W, f_b, cls_W, cls_b):
    node_out, c_vec = pl.pallas_call(
        _prep_body,
        out_shape=[jax.ShapeDtypeStruct((8, N_NODES), jnp.float32),
                   jax.ShapeDtypeStruct((EA, 1), jnp.float32)],
    )(x_a, g_W, g_b, f_W, f_b, cls_W, cls_b)

    eaT = edge_attr_a2a.T
    EB = N_EDGES // 10
    r2, src2, dst2 = pl.pallas_call(
        _edge_body,
        grid=(10,),
        in_specs=[pl.BlockSpec((EA, EB), lambda i: (0, i)),
                  pl.BlockSpec((2, EB), lambda i: (0, i)),
                  pl.BlockSpec((EA, 1), lambda i: (0, 0))],
        out_specs=[pl.BlockSpec((1, EB), lambda i: (0, i)),
                   pl.BlockSpec((1, EB), lambda i: (0, i)),
                   pl.BlockSpec((1, EB), lambda i: (0, i))],
        out_shape=[jax.ShapeDtypeStruct((1, N_EDGES), jnp.float32),
                   jax.ShapeDtypeStruct((1, N_EDGES), jnp.int32),
                   jax.ShapeDtypeStruct((1, N_EDGES), jnp.int32)],
    )(eaT, edge_index_a2a, c_vec)
    r = r2.reshape(-1)
    src = src2.reshape(-1)
    dst = dst2.reshape(-1)
    p = node_out[2]

    T_parts, cnt_parts = _segment_parts(src, dst, r, p)

    out_row = pl.pallas_call(
        _combine_body,
        out_shape=jax.ShapeDtypeStruct((1, N_NODES), jnp.float32),
    )(node_out, T_parts, cnt_parts)
    return out_row.reshape(N_NODES, 1)


# speedup vs baseline: 1.3103x; 1.3103x over previous
import functools

import jax
import jax.numpy as jnp
from jax import lax
from jax.experimental import pallas as pl
from jax.experimental.pallas import tpu as pltpu
from jax.experimental.pallas import tpu_sc as plsc

N_NODES = 10000
N_EDGES = 320000
CH = 128
EA = 16

NC = 2
NS = 16
L = 16
NW = NC * NS
EPAD = 327680
EPW = EPAD // NW
STEPS = EPW // L
NPAD = N_NODES + L
ZSTEPS = NPAD // L

_HI = lax.Precision.HIGHEST


def _prep_body(x_ref, gW_ref, gb_ref, fW_ref, fb_ref, cW_ref, cb_ref,
               node_ref, c_ref):
    cW = cW_ref[...]
    fW = fW_ref[...]
    u = lax.dot_general(fW[0:CH], cW, (((1,), (0,)), ((), ())), precision=_HI)
    v = lax.dot_general(fW[CH:2 * CH], cW, (((1,), (0,)), ((), ())), precision=_HI)
    gW = gW_ref[...]
    a = lax.dot_general(gW[0:CH], v, (((1,), (0,)), ((), ())), precision=_HI)
    b = lax.dot_general(gW[CH:2 * CH], v, (((1,), (0,)), ((), ())), precision=_HI)
    c = lax.dot_general(gW[2 * CH:2 * CH + EA], v, (((1,), (0,)), ((), ())), precision=_HI)
    s0 = jnp.sum(gb_ref[...] * v[:, 0])
    s1 = jnp.sum(fb_ref[...] * cW[:, 0]) + jnp.sum(cb_ref[...])
    W3 = jnp.concatenate([u, a, b, jnp.zeros((CH, 5), jnp.float32)], axis=1)
    node = lax.dot_general(W3, x_ref[...], (((0,), (1,)), ((), ())), precision=_HI)
    row = lax.broadcasted_iota(jnp.int32, node.shape, 0)
    node_ref[...] = node + jnp.where(row == 0, s1, 0.0) + jnp.where(row == 1, s0, 0.0)
    c_ref[...] = c


def _edge_body(X_ref, ei_ref, c_ref, r_ref, src_ref, dst_ref):
    i = pl.program_id(0)
    eb = r_ref.shape[1]
    e_idx = i * eb + lax.broadcasted_iota(jnp.int32, (1, eb), 1)
    valid = e_idx < N_EDGES
    acc = X_ref[0:1, :] * c_ref[0:1, 0:1]
    for j in range(1, EA):
        acc = acc + X_ref[j:j + 1, :] * c_ref[j:j + 1, 0:1]
    r_ref[...] = jnp.where(valid, acc, 0.0)
    src_ref[...] = jnp.where(valid, ei_ref[0:1, :], 0)
    dst_ref[...] = jnp.where(valid, ei_ref[1:2, :], N_NODES)


def _combine_body(node_ref, T_ref, cnt_ref, out_ref):
    T = jnp.sum(T_ref[:, 0:N_NODES], axis=0, keepdims=True)
    cnt = jnp.sum(cnt_ref[:, 0:N_NODES], axis=0, keepdims=True)
    xu = node_ref[0:1, :]
    xa = node_ref[1:2, :]
    out_ref[...] = xu + jnp.where(cnt > 0.0, xa + T / jnp.maximum(cnt, 1.0), 0.0)


@functools.cache
def _sc_segsum_kernel():
    return pl.kernel(
        _sc_segsum_body,
        out_type=[jax.ShapeDtypeStruct((NW, NPAD), jnp.float32),
                  jax.ShapeDtypeStruct((NW, NPAD), jnp.float32)],
        mesh=plsc.VectorSubcoreMesh(core_axis_name="c", subcore_axis_name="s",
                                    num_cores=NC, num_subcores=NS),
        compiler_params=pltpu.CompilerParams(needs_layout_passes=False),
        scratch_types=[pltpu.VMEM((EPW,), jnp.int32),
                       pltpu.VMEM((EPW,), jnp.int32),
                       pltpu.VMEM((EPW,), jnp.float32),
                       pltpu.VMEM((N_NODES,), jnp.float32),
                       pltpu.VMEM((NPAD,), jnp.float32),
                       pltpu.VMEM((NPAD,), jnp.float32)],
    )


def _sc_segsum_body(src_hbm, dst_hbm, r_hbm, p_hbm, T_hbm, cnt_hbm,
                    src_v, dst_v, r_v, p_v, T_v, cnt_v):
    wid = lax.axis_index("s") * NC + lax.axis_index("c")
    base = pl.multiple_of(wid * EPW, 128)
    pltpu.sync_copy(src_hbm.at[0, pl.ds(base, EPW)], src_v)
    pltpu.sync_copy(dst_hbm.at[0, pl.ds(base, EPW)], dst_v)
    pltpu.sync_copy(r_hbm.at[0, pl.ds(base, EPW)], r_v)
    pltpu.sync_copy(p_hbm, p_v)

    def zero_body(i, carry):
        off = i * L
        z = jnp.zeros((L,), jnp.float32)
        T_v[pl.ds(off, L)] = z
        cnt_v[pl.ds(off, L)] = z
        return carry

    lax.fori_loop(0, ZSTEPS, zero_body, 0)

    def body(i, carry):
        off = i * L
        s = src_v[pl.ds(off, L)]
        d = dst_v[pl.ds(off, L)]
        rv = r_v[pl.ds(off, L)]
        pv = plsc.load_gather(p_v, [s])
        plsc.addupdate_scatter(T_v, [d], pv + rv)
        plsc.addupdate_scatter(cnt_v, [d], jnp.full((L,), 1.0, jnp.float32))
        return carry

    lax.fori_loop(0, STEPS, body, 0)

    pltpu.sync_copy(T_v, T_hbm.at[wid])
    pltpu.sync_copy(cnt_v, cnt_hbm.at[wid])


def _segment_parts(src, dst, r, p):
    return _sc_segsum_kernel()(src, dst, r, p)


def kernel(x_a, edge_index_a2a, edge_attr_a2a, g_W, g_b, f_W, f_b, cls_W, cls_b):
    node_out, c_vec = pl.pallas_call(
        _prep_body,
        out_shape=[jax.ShapeDtypeStruct((8, N_NODES), jnp.float32),
                   jax.ShapeDtypeStruct((EA, 1), jnp.float32)],
    )(x_a, g_W, g_b, f_W, f_b, cls_W, cls_b)

    eaT = edge_attr_a2a.T
    EB = EPAD // 10
    r2, src2, dst2 = pl.pallas_call(
        _edge_body,
        grid=(10,),
        in_specs=[pl.BlockSpec((EA, EB), lambda i: (0, i)),
                  pl.BlockSpec((2, EB), lambda i: (0, i)),
                  pl.BlockSpec((EA, 1), lambda i: (0, 0))],
        out_specs=[pl.BlockSpec((1, EB), lambda i: (0, i)),
                   pl.BlockSpec((1, EB), lambda i: (0, i)),
                   pl.BlockSpec((1, EB), lambda i: (0, i))],
        out_shape=[jax.ShapeDtypeStruct((1, EPAD), jnp.float32),
                   jax.ShapeDtypeStruct((1, EPAD), jnp.int32),
                   jax.ShapeDtypeStruct((1, EPAD), jnp.int32)],
    )(eaT, edge_index_a2a, c_vec)
    p = node_out[2]

    T_parts, cnt_parts = _segment_parts(src2, dst2, r2, p)

    out_row = pl.pallas_call(
        _combine_body,
        out_shape=jax.ShapeDtypeStruct((1, N_NODES), jnp.float32),
    )(node_out, T_parts, cnt_parts)
    return out_row.reshape(N_NODES, 1)


# speedup vs baseline: 1.3635x; 1.0406x over previous
import functools

import jax
import jax.numpy as jnp
from jax import lax
from jax.experimental import pallas as pl
from jax.experimental.pallas import tpu as pltpu
from jax.experimental.pallas import tpu_sc as plsc

N_NODES = 10000
N_EDGES = 320000
CH = 128
EA = 16

NC = 2
NS = 16
L = 16
NW = NC * NS
EPAD = 327680
EPW = EPAD // NW
STEPS = EPW // L
NPAD = 10240
ZSTEPS = NPAD // L
IDB = 14

_HI = lax.Precision.HIGHEST


def _prep_body(x_ref, gW_ref, gb_ref, fW_ref, fb_ref, cW_ref, cb_ref,
               node_ref, c_ref):
    cW = cW_ref[...]
    fW = fW_ref[...]
    u = lax.dot_general(fW[0:CH], cW, (((1,), (0,)), ((), ())), precision=_HI)
    v = lax.dot_general(fW[CH:2 * CH], cW, (((1,), (0,)), ((), ())), precision=_HI)
    gW = gW_ref[...]
    a = lax.dot_general(gW[0:CH], v, (((1,), (0,)), ((), ())), precision=_HI)
    b = lax.dot_general(gW[CH:2 * CH], v, (((1,), (0,)), ((), ())), precision=_HI)
    c = lax.dot_general(gW[2 * CH:2 * CH + EA], v, (((1,), (0,)), ((), ())), precision=_HI)
    s0 = jnp.sum(gb_ref[...] * v[:, 0])
    s1 = jnp.sum(fb_ref[...] * cW[:, 0]) + jnp.sum(cb_ref[...])
    W3 = jnp.concatenate([u, a, b, jnp.zeros((CH, 5), jnp.float32)], axis=1)
    node = lax.dot_general(W3, x_ref[...], (((0,), (1,)), ((), ())), precision=_HI)
    row = lax.broadcasted_iota(jnp.int32, node.shape, 0)
    node_ref[...] = node + jnp.where(row == 0, s1, 0.0) + jnp.where(row == 1, s0, 0.0)
    c_ref[...] = c


def _edge_body(X_ref, ei_ref, c_ref, r_ref, sd_ref):
    i = pl.program_id(0)
    eb = r_ref.shape[1]
    e_idx = i * eb + lax.broadcasted_iota(jnp.int32, (1, eb), 1)
    valid = e_idx < N_EDGES
    acc = X_ref[0:1, :] * c_ref[0:1, 0:1]
    for j in range(1, EA):
        acc = acc + X_ref[j:j + 1, :] * c_ref[j:j + 1, 0:1]
    r_ref[...] = jnp.where(valid, acc, 0.0)
    packed = jnp.where(valid,
                       lax.shift_left(ei_ref[1:2, :], IDB) | ei_ref[0:1, :],
                       N_NODES << IDB)
    sd_ref[...] = packed


def _combine_body(node_ref, T_ref, cnt_ref, out_ref):
    T = jnp.sum(T_ref[:, 0:N_NODES], axis=0, keepdims=True)
    cnt = jnp.sum(cnt_ref[:, 0:N_NODES], axis=0, keepdims=True)
    xu = node_ref[0:1, :]
    xa = node_ref[1:2, :]
    out_ref[...] = xu + jnp.where(cnt > 0.0, xa + T / jnp.maximum(cnt, 1.0), 0.0)


@functools.cache
def _sc_segsum_kernel():
    return pl.kernel(
        _sc_segsum_body,
        out_type=[jax.ShapeDtypeStruct((NW, NPAD), jnp.float32),
                  jax.ShapeDtypeStruct((NW, NPAD), jnp.float32)],
        mesh=plsc.VectorSubcoreMesh(core_axis_name="c", subcore_axis_name="s",
                                    num_cores=NC, num_subcores=NS),
        compiler_params=pltpu.CompilerParams(needs_layout_passes=False),
        scratch_types=[pltpu.VMEM((EPW,), jnp.int32),
                       pltpu.VMEM((EPW,), jnp.float32),
                       pltpu.VMEM((N_NODES,), jnp.float32),
                       pltpu.VMEM((NPAD,), jnp.float32),
                       pltpu.VMEM((NPAD,), jnp.float32),
                       pltpu.SemaphoreType.DMA,
                       pltpu.SemaphoreType.DMA,
                       pltpu.SemaphoreType.DMA],
    )


_UNROLL = 4


def _sc_segsum_body(sd_hbm, r_hbm, p_hbm, T_hbm, cnt_hbm,
                    sd_v, r_v, p_v, T_v, cnt_v, sem0, sem1, sem2):
    wid = lax.axis_index("s") * NC + lax.axis_index("c")
    base = pl.multiple_of(wid * EPW, 128)
    cp0 = pltpu.async_copy(sd_hbm.at[0, pl.ds(base, EPW)], sd_v, sem0)
    cp1 = pltpu.async_copy(r_hbm.at[0, pl.ds(base, EPW)], r_v, sem1)
    cp2 = pltpu.async_copy(p_hbm, p_v, sem2)

    def zero_body(i, carry):
        off = i * (L * _UNROLL)
        z = jnp.zeros((L,), jnp.float32)
        for u in range(_UNROLL):
            T_v[pl.ds(off + u * L, L)] = z
            cnt_v[pl.ds(off + u * L, L)] = z
        return carry

    lax.fori_loop(0, ZSTEPS // _UNROLL, zero_body, 0)
    cp0.wait()
    cp1.wait()
    cp2.wait()

    mask = jnp.full((L,), (1 << IDB) - 1, jnp.int32)
    ones = jnp.full((L,), 1.0, jnp.float32)

    def body(i, carry):
        off = i * (L * _UNROLL)
        for u in range(_UNROLL):
            sd = sd_v[pl.ds(off + u * L, L)]
            rv = r_v[pl.ds(off + u * L, L)]
            s = sd & mask
            d = lax.shift_right_logical(sd, IDB)
            pv = plsc.load_gather(p_v, [s])
            plsc.addupdate_scatter(T_v, [d], pv + rv)
            plsc.addupdate_scatter(cnt_v, [d], ones)
        return carry

    lax.fori_loop(0, STEPS // _UNROLL, body, 0)

    pltpu.sync_copy(T_v, T_hbm.at[wid])
    pltpu.sync_copy(cnt_v, cnt_hbm.at[wid])


def _segment_parts(sd, r, p):
    return _sc_segsum_kernel()(sd, r, p)


def kernel(x_a, edge_index_a2a, edge_attr_a2a, g_W, g_b, f_W, f_b, cls_W, cls_b):
    node_out, c_vec = pl.pallas_call(
        _prep_body,
        out_shape=[jax.ShapeDtypeStruct((8, N_NODES), jnp.float32),
                   jax.ShapeDtypeStruct((EA, 1), jnp.float32)],
    )(x_a, g_W, g_b, f_W, f_b, cls_W, cls_b)

    eaT = edge_attr_a2a.T
    EB = EPAD // 10
    r2, sd2 = pl.pallas_call(
        _edge_body,
        grid=(10,),
        in_specs=[pl.BlockSpec((EA, EB), lambda i: (0, i)),
                  pl.BlockSpec((2, EB), lambda i: (0, i)),
                  pl.BlockSpec((EA, 1), lambda i: (0, 0))],
        out_specs=[pl.BlockSpec((1, EB), lambda i: (0, i)),
                   pl.BlockSpec((1, EB), lambda i: (0, i))],
        out_shape=[jax.ShapeDtypeStruct((1, EPAD), jnp.float32),
                   jax.ShapeDtypeStruct((1, EPAD), jnp.int32)],
    )(eaT, edge_index_a2a, c_vec)
    p = node_out[2]

    T_parts, cnt_parts = _segment_parts(sd2, r2, p)

    out_row = pl.pallas_call(
        _combine_body,
        out_shape=jax.ShapeDtypeStruct((1, N_NODES), jnp.float32),
    )(node_out, T_parts, cnt_parts)
    return out_row.reshape(N_NODES, 1)


# speedup vs baseline: 1.5070x; 1.1053x over previous
import functools

import jax
import jax.numpy as jnp
from jax import lax
from jax.experimental import pallas as pl
from jax.experimental.pallas import tpu as pltpu
from jax.experimental.pallas import tpu_sc as plsc

N_NODES = 10000
N_EDGES = 320000
CH = 128
EA = 16

NC = 2
NS = 16
L = 16
NW = NC * NS
EPAD = 327680
PAIR = EPAD // NS
EPW0 = 14720
EPW1 = PAIR - EPW0
NPAD = 10240
ZSTEPS = NPAD // L
IDB = 14

_HI = lax.Precision.HIGHEST


def _prep_body(x_ref, gW_ref, gb_ref, fW_ref, fb_ref, cW_ref, cb_ref,
               node_ref, c_ref):
    cW = cW_ref[...]
    fW = fW_ref[...]
    u = lax.dot_general(fW[0:CH], cW, (((1,), (0,)), ((), ())), precision=_HI)
    v = lax.dot_general(fW[CH:2 * CH], cW, (((1,), (0,)), ((), ())), precision=_HI)
    gW = gW_ref[...]
    a = lax.dot_general(gW[0:CH], v, (((1,), (0,)), ((), ())), precision=_HI)
    b = lax.dot_general(gW[CH:2 * CH], v, (((1,), (0,)), ((), ())), precision=_HI)
    c = lax.dot_general(gW[2 * CH:2 * CH + EA], v, (((1,), (0,)), ((), ())), precision=_HI)
    s0 = jnp.sum(gb_ref[...] * v[:, 0])
    s1 = jnp.sum(fb_ref[...] * cW[:, 0]) + jnp.sum(cb_ref[...])
    W3 = jnp.concatenate([u, a, b, jnp.zeros((CH, 5), jnp.float32)], axis=1)
    node = lax.dot_general(W3, x_ref[...], (((0,), (1,)), ((), ())), precision=_HI)
    row = lax.broadcasted_iota(jnp.int32, node.shape, 0)
    node_ref[...] = node + jnp.where(row == 0, s1, 0.0) + jnp.where(row == 1, s0, 0.0)
    c_ref[...] = c


def _edge_body(X_ref, ei_ref, c_ref, r_ref, sd_ref):
    i = pl.program_id(0)
    eb = r_ref.shape[1]
    e_idx = i * eb + lax.broadcasted_iota(jnp.int32, (1, eb), 1)
    valid = e_idx < N_EDGES
    acc = X_ref[0:1, :] * c_ref[0:1, 0:1]
    for j in range(1, EA):
        acc = acc + X_ref[j:j + 1, :] * c_ref[j:j + 1, 0:1]
    r_ref[...] = jnp.where(valid, acc, 0.0)
    packed = jnp.where(valid,
                       lax.shift_left(ei_ref[1:2, :], IDB) | ei_ref[0:1, :],
                       N_NODES << IDB)
    sd_ref[...] = packed


def _combine_body(node_ref, T_ref, cnt_ref, out_ref):
    T = jnp.sum(T_ref[:, 0:N_NODES], axis=0, keepdims=True)
    cnt = jnp.sum(cnt_ref[:, 0:N_NODES], axis=0, keepdims=True)
    xu = node_ref[0:1, :]
    xa = node_ref[1:2, :]
    out_ref[...] = xu + jnp.where(cnt > 0.0, xa + T / jnp.maximum(cnt, 1.0), 0.0)


@functools.cache
def _sc_segsum_kernel():
    return pl.kernel(
        _sc_segsum_body,
        out_type=[jax.ShapeDtypeStruct((NW, NPAD), jnp.float32),
                  jax.ShapeDtypeStruct((NW, NPAD), jnp.float32)],
        mesh=plsc.VectorSubcoreMesh(core_axis_name="c", subcore_axis_name="s",
                                    num_cores=NC, num_subcores=NS),
        compiler_params=pltpu.CompilerParams(needs_layout_passes=False),
        scratch_types=[pltpu.VMEM((EPW0,), jnp.int32),
                       pltpu.VMEM((EPW0,), jnp.float32),
                       pltpu.VMEM((N_NODES,), jnp.float32),
                       pltpu.VMEM((NPAD,), jnp.float32),
                       pltpu.VMEM((NPAD,), jnp.float32),
                       pltpu.SemaphoreType.DMA,
                       pltpu.SemaphoreType.DMA,
                       pltpu.SemaphoreType.DMA],
    )


_UNROLL = 4


def _sc_segsum_body(sd_hbm, r_hbm, p_hbm, T_hbm, cnt_hbm,
                    sd_v, r_v, p_v, T_v, cnt_v, sem0, sem1, sem2):
    c = lax.axis_index("c")
    s = lax.axis_index("s")
    wid = s * NC + c

    def run(nedges, base):
        cp0 = pltpu.async_copy(sd_hbm.at[0, pl.ds(base, nedges)],
                               sd_v.at[pl.ds(0, nedges)], sem0)
        cp1 = pltpu.async_copy(r_hbm.at[0, pl.ds(base, nedges)],
                               r_v.at[pl.ds(0, nedges)], sem1)
        cp2 = pltpu.async_copy(p_hbm, p_v, sem2)

        def zero_body(i, carry):
            off = i * (L * _UNROLL)
            z = jnp.zeros((L,), jnp.float32)
            for u in range(_UNROLL):
                T_v[pl.ds(off + u * L, L)] = z
                cnt_v[pl.ds(off + u * L, L)] = z
            return carry

        lax.fori_loop(0, ZSTEPS // _UNROLL, zero_body, 0)
        cp0.wait()
        cp1.wait()
        cp2.wait()

        mask = jnp.full((L,), (1 << IDB) - 1, jnp.int32)
        ones = jnp.full((L,), 1.0, jnp.float32)

        def body(i, carry):
            off = i * (L * _UNROLL)
            for u in range(_UNROLL):
                sd = sd_v[pl.ds(off + u * L, L)]
                rv = r_v[pl.ds(off + u * L, L)]
                sidx = sd & mask
                d = lax.shift_right_logical(sd, IDB)
                pv = plsc.load_gather(p_v, [sidx])
                plsc.addupdate_scatter(T_v, [d], pv + rv)
                plsc.addupdate_scatter(cnt_v, [d], ones)
            return carry

        lax.fori_loop(0, nedges // (L * _UNROLL), body, 0)

    @pl.when(c == 0)
    def _():
        run(EPW0, pl.multiple_of(s * PAIR, 128))

    @pl.when(c == 1)
    def _():
        run(EPW1, pl.multiple_of(s * PAIR + EPW0, 128))

    pltpu.sync_copy(T_v, T_hbm.at[wid])
    pltpu.sync_copy(cnt_v, cnt_hbm.at[wid])


def _segment_parts(sd, r, p):
    return _sc_segsum_kernel()(sd, r, p)


def kernel(x_a, edge_index_a2a, edge_attr_a2a, g_W, g_b, f_W, f_b, cls_W, cls_b):
    node_out, c_vec = pl.pallas_call(
        _prep_body,
        out_shape=[jax.ShapeDtypeStruct((8, N_NODES), jnp.float32),
                   jax.ShapeDtypeStruct((EA, 1), jnp.float32)],
    )(x_a, g_W, g_b, f_W, f_b, cls_W, cls_b)

    eaT = edge_attr_a2a.T
    EB = EPAD // 10
    r2, sd2 = pl.pallas_call(
        _edge_body,
        grid=(10,),
        in_specs=[pl.BlockSpec((EA, EB), lambda i: (0, i)),
                  pl.BlockSpec((2, EB), lambda i: (0, i)),
                  pl.BlockSpec((EA, 1), lambda i: (0, 0))],
        out_specs=[pl.BlockSpec((1, EB), lambda i: (0, i)),
                   pl.BlockSpec((1, EB), lambda i: (0, i))],
        out_shape=[jax.ShapeDtypeStruct((1, EPAD), jnp.float32),
                   jax.ShapeDtypeStruct((1, EPAD), jnp.int32)],
    )(eaT, edge_index_a2a, c_vec)
    p = node_out[2]

    T_parts, cnt_parts = _segment_parts(sd2, r2, p)

    out_row = pl.pallas_call(
        _combine_body,
        out_shape=jax.ShapeDtypeStruct((1, N_NODES), jnp.float32),
    )(node_out, T_parts, cnt_parts)
    return out_row.reshape(N_NODES, 1)


# speedup vs baseline: 1.5344x; 1.0182x over previous
import functools

import jax
import jax.numpy as jnp
from jax import lax
from jax.experimental import pallas as pl
from jax.experimental.pallas import tpu as pltpu
from jax.experimental.pallas import tpu_sc as plsc

N_NODES = 10000
N_EDGES = 320000
CH = 128
EA = 16

NC = 2
NS = 16
L = 16
NW = NC * NS
EPAD = 327680
PAIR = EPAD // NS
EPW0 = 14720
EPW1 = PAIR - EPW0
NPAD = 10240
ZSTEPS = NPAD // L
IDB = 14

_HI = lax.Precision.HIGHEST


def _prep_body(x_ref, gW_ref, gb_ref, fW_ref, fb_ref, cW_ref, cb_ref,
               node_ref, c_ref):
    cW = cW_ref[...]
    fW = fW_ref[...]
    u = lax.dot_general(fW[0:CH], cW, (((1,), (0,)), ((), ())), precision=_HI)
    v = lax.dot_general(fW[CH:2 * CH], cW, (((1,), (0,)), ((), ())), precision=_HI)
    gW = gW_ref[...]
    a = lax.dot_general(gW[0:CH], v, (((1,), (0,)), ((), ())), precision=_HI)
    b = lax.dot_general(gW[CH:2 * CH], v, (((1,), (0,)), ((), ())), precision=_HI)
    c = lax.dot_general(gW[2 * CH:2 * CH + EA], v, (((1,), (0,)), ((), ())), precision=_HI)
    s0 = jnp.sum(gb_ref[...] * v[:, 0])
    s1 = jnp.sum(fb_ref[...] * cW[:, 0]) + jnp.sum(cb_ref[...])
    W3 = jnp.concatenate([u, a, b, jnp.zeros((CH, 5), jnp.float32)], axis=1)
    node = lax.dot_general(W3, x_ref[...], (((0,), (1,)), ((), ())), precision=_HI)
    row = lax.broadcasted_iota(jnp.int32, node.shape, 0)
    node_ref[...] = node + jnp.where(row == 0, s1, 0.0) + jnp.where(row == 1, s0, 0.0)
    c_ref[...] = c


def _edge_body(X_ref, ei_ref, c_ref, r_ref, sd_ref):
    i = pl.program_id(0)
    eb = r_ref.shape[1]
    e_idx = i * eb + lax.broadcasted_iota(jnp.int32, (1, eb), 1)
    valid = e_idx < N_EDGES
    acc = jnp.sum(X_ref[...] * c_ref[...], axis=0, keepdims=True)
    r_ref[...] = jnp.where(valid, acc, 0.0)
    packed = jnp.where(valid,
                       lax.shift_left(ei_ref[1:2, :], IDB) | ei_ref[0:1, :],
                       N_NODES << IDB)
    sd_ref[...] = packed


def _combine_body(node_ref, T_ref, cnt_ref, out_ref):
    T = jnp.sum(T_ref[:, 0:N_NODES], axis=0, keepdims=True)
    cnt = jnp.sum(cnt_ref[:, 0:N_NODES], axis=0, keepdims=True)
    xu = node_ref[0:1, :]
    xa = node_ref[1:2, :]
    out_ref[...] = xu + jnp.where(cnt > 0.0, xa + T / jnp.maximum(cnt, 1.0), 0.0)


@functools.cache
def _sc_segsum_kernel():
    return pl.kernel(
        _sc_segsum_body,
        out_type=[jax.ShapeDtypeStruct((NW, NPAD), jnp.float32),
                  jax.ShapeDtypeStruct((NW, NPAD), jnp.float32)],
        mesh=plsc.VectorSubcoreMesh(core_axis_name="c", subcore_axis_name="s",
                                    num_cores=NC, num_subcores=NS),
        compiler_params=pltpu.CompilerParams(needs_layout_passes=False),
        scratch_types=[pltpu.VMEM((EPW0,), jnp.int32),
                       pltpu.VMEM((EPW0,), jnp.float32),
                       pltpu.VMEM((N_NODES,), jnp.float32),
                       pltpu.VMEM((NPAD,), jnp.float32),
                       pltpu.VMEM((NPAD,), jnp.float32),
                       pltpu.SemaphoreType.DMA,
                       pltpu.SemaphoreType.DMA,
                       pltpu.SemaphoreType.DMA],
    )


_UNROLL = 4


def _sc_segsum_body(sd_hbm, r_hbm, p_hbm, T_hbm, cnt_hbm,
                    sd_v, r_v, p_v, T_v, cnt_v, sem0, sem1, sem2):
    c = lax.axis_index("c")
    s = lax.axis_index("s")
    wid = s * NC + c

    def run(nedges, base):
        cp0 = pltpu.async_copy(sd_hbm.at[0, pl.ds(base, nedges)],
                               sd_v.at[pl.ds(0, nedges)], sem0)
        cp1 = pltpu.async_copy(r_hbm.at[0, pl.ds(base, nedges)],
                               r_v.at[pl.ds(0, nedges)], sem1)
        cp2 = pltpu.async_copy(p_hbm, p_v, sem2)

        def zero_body(i, carry):
            off = i * (L * _UNROLL)
            z = jnp.zeros((L,), jnp.float32)
            for u in range(_UNROLL):
                T_v[pl.ds(off + u * L, L)] = z
                cnt_v[pl.ds(off + u * L, L)] = z
            return carry

        lax.fori_loop(0, ZSTEPS // _UNROLL, zero_body, 0)
        cp0.wait()
        cp1.wait()
        cp2.wait()

        mask = jnp.full((L,), (1 << IDB) - 1, jnp.int32)
        ones = jnp.full((L,), 1.0, jnp.float32)

        def body(i, carry):
            off = i * (L * _UNROLL)
            for u in range(_UNROLL):
                sd = sd_v[pl.ds(off + u * L, L)]
                rv = r_v[pl.ds(off + u * L, L)]
                sidx = sd & mask
                d = lax.shift_right_logical(sd, IDB)
                pv = plsc.load_gather(p_v, [sidx])
                plsc.addupdate_scatter(T_v, [d], pv + rv)
                plsc.addupdate_scatter(cnt_v, [d], ones)
            return carry

        lax.fori_loop(0, nedges // (L * _UNROLL), body, 0)

    @pl.when(c == 0)
    def _():
        run(EPW0, pl.multiple_of(s * PAIR, 128))

    @pl.when(c == 1)
    def _():
        run(EPW1, pl.multiple_of(s * PAIR + EPW0, 128))

    pltpu.sync_copy(T_v, T_hbm.at[wid])
    pltpu.sync_copy(cnt_v, cnt_hbm.at[wid])


def _segment_parts(sd, r, p):
    return _sc_segsum_kernel()(sd, r, p)


def kernel(x_a, edge_index_a2a, edge_attr_a2a, g_W, g_b, f_W, f_b, cls_W, cls_b):
    node_out, c_vec = pl.pallas_call(
        _prep_body,
        out_shape=[jax.ShapeDtypeStruct((8, N_NODES), jnp.float32),
                   jax.ShapeDtypeStruct((EA, 1), jnp.float32)],
    )(x_a, g_W, g_b, f_W, f_b, cls_W, cls_b)

    eaT = edge_attr_a2a.T
    EB = EPAD // 10
    r2, sd2 = pl.pallas_call(
        _edge_body,
        grid=(10,),
        in_specs=[pl.BlockSpec((EA, EB), lambda i: (0, i)),
                  pl.BlockSpec((2, EB), lambda i: (0, i)),
                  pl.BlockSpec((EA, 1), lambda i: (0, 0))],
        out_specs=[pl.BlockSpec((1, EB), lambda i: (0, i)),
                   pl.BlockSpec((1, EB), lambda i: (0, i))],
        out_shape=[jax.ShapeDtypeStruct((1, EPAD), jnp.float32),
                   jax.ShapeDtypeStruct((1, EPAD), jnp.int32)],
    )(eaT, edge_index_a2a, c_vec)
    p = node_out[2]

    T_parts, cnt_parts = _segment_parts(sd2, r2, p)

    out_row = pl.pallas_call(
        _combine_body,
        out_shape=jax.ShapeDtypeStruct((1, N_NODES), jnp.float32),
    )(node_out, T_parts, cnt_parts)
    return out_row.reshape(N_NODES, 1)


# speedup vs baseline: 1.6238x; 1.0583x over previous
import functools

import jax
import jax.numpy as jnp
from jax import lax
from jax.experimental import pallas as pl
from jax.experimental.pallas import tpu as pltpu
from jax.experimental.pallas import tpu_sc as plsc

N_NODES = 10000
N_EDGES = 320000
CH = 128
EA = 16

NC = 2
NS = 16
L = 16
NW = NC * NS
EPAD = 327680
PAIR = EPAD // NS
EPW0 = 14720
EPW1 = PAIR - EPW0
NPAD = 10240
ZSTEPS = NPAD // L
IDB = 14

_HI = lax.Precision.HIGHEST


def _prep_body(x_ref, gW_ref, gb_ref, fW_ref, fb_ref, cW_ref, cb_ref,
               node_ref, c_ref):
    cW = cW_ref[...]
    fW = fW_ref[...]
    u = lax.dot_general(fW[0:CH], cW, (((1,), (0,)), ((), ())), precision=_HI)
    v = lax.dot_general(fW[CH:2 * CH], cW, (((1,), (0,)), ((), ())), precision=_HI)
    gW = gW_ref[...]
    a = lax.dot_general(gW[0:CH], v, (((1,), (0,)), ((), ())), precision=_HI)
    b = lax.dot_general(gW[CH:2 * CH], v, (((1,), (0,)), ((), ())), precision=_HI)
    c = lax.dot_general(gW[2 * CH:2 * CH + EA], v, (((1,), (0,)), ((), ())), precision=_HI)
    s0 = jnp.sum(gb_ref[...] * v[:, 0])
    s1 = jnp.sum(fb_ref[...] * cW[:, 0]) + jnp.sum(cb_ref[...])
    W3 = jnp.concatenate([u, a, b, jnp.zeros((CH, 5), jnp.float32)], axis=1)
    node = lax.dot_general(W3, x_ref[...], (((0,), (1,)), ((), ())), precision=_HI)
    row = lax.broadcasted_iota(jnp.int32, node.shape, 0)
    node_ref[...] = node + jnp.where(row == 0, s1, 0.0) + jnp.where(row == 1, s0, 0.0)
    c_ref[...] = c


def _edge_body(X_ref, ei_ref, c_ref, r_ref, sd_ref):
    i = pl.program_id(0)
    eb = r_ref.shape[1]
    e_idx = i * eb + lax.broadcasted_iota(jnp.int32, (1, eb), 1)
    valid = e_idx < N_EDGES
    acc = jnp.sum(X_ref[...] * c_ref[...], axis=0, keepdims=True)
    r_ref[...] = jnp.where(valid, acc, 0.0)
    packed = jnp.where(valid,
                       lax.shift_left(ei_ref[1:2, :], IDB) | ei_ref[0:1, :],
                       N_NODES << IDB)
    sd_ref[...] = packed


def _combine_body(node_ref, T_ref, cnt_ref, out_ref):
    T = jnp.sum(T_ref[:, 0:N_NODES], axis=0, keepdims=True)
    cnt = jnp.sum(cnt_ref[:, 0:N_NODES], axis=0, keepdims=True)
    xu = node_ref[0:1, :]
    xa = node_ref[1:2, :]
    out_ref[...] = xu + jnp.where(cnt > 0.0, xa + T / jnp.maximum(cnt, 1.0), 0.0)


@functools.cache
def _sc_segsum_kernel():
    return pl.kernel(
        _sc_segsum_body,
        out_type=[jax.ShapeDtypeStruct((NW, NPAD), jnp.float32),
                  jax.ShapeDtypeStruct((NW, NPAD), jnp.float32)],
        mesh=plsc.VectorSubcoreMesh(core_axis_name="c", subcore_axis_name="s",
                                    num_cores=NC, num_subcores=NS),
        compiler_params=pltpu.CompilerParams(needs_layout_passes=False),
        scratch_types=[pltpu.VMEM((EPW0,), jnp.int32),
                       pltpu.VMEM((EPW0,), jnp.float32),
                       pltpu.VMEM((N_NODES,), jnp.float32),
                       pltpu.VMEM((NPAD,), jnp.float32),
                       pltpu.VMEM((NPAD,), jnp.float32),
                       pltpu.SemaphoreType.DMA,
                       pltpu.SemaphoreType.DMA,
                       pltpu.SemaphoreType.DMA],
    )


_UNROLL = 4


def _sc_segsum_body(sd_hbm, r_hbm, p_hbm, T_hbm, cnt_hbm,
                    sd_v, r_v, p_v, T_v, cnt_v, sem0, sem1, sem2):
    c = lax.axis_index("c")
    s = lax.axis_index("s")
    wid = s * NC + c

    def run(nedges, base):
        cp0 = pltpu.async_copy(sd_hbm.at[0, pl.ds(base, nedges)],
                               sd_v.at[pl.ds(0, nedges)], sem0)
        cp1 = pltpu.async_copy(r_hbm.at[0, pl.ds(base, nedges)],
                               r_v.at[pl.ds(0, nedges)], sem1)
        cp2 = pltpu.async_copy(p_hbm, p_v, sem2)

        def zero_body(i, carry):
            off = i * (L * _UNROLL)
            z = jnp.zeros((L,), jnp.float32)
            for u in range(_UNROLL):
                T_v[pl.ds(off + u * L, L)] = z
                cnt_v[pl.ds(off + u * L, L)] = z
            return carry

        lax.fori_loop(0, ZSTEPS // _UNROLL, zero_body, 0)
        cp0.wait()
        cp1.wait()
        cp2.wait()

        mask = jnp.full((L,), (1 << IDB) - 1, jnp.int32)
        ones = jnp.full((L,), 1.0, jnp.float32)

        @plsc.parallel_loop(0, nedges // L, 1, unroll=_UNROLL)
        def body(i):
            off = i * L
            sd = sd_v[pl.ds(off, L)]
            rv = r_v[pl.ds(off, L)]
            sidx = sd & mask
            d = lax.shift_right_logical(sd, IDB)
            pv = plsc.load_gather(p_v, [sidx])
            plsc.addupdate_scatter(T_v, [d], pv + rv)
            plsc.addupdate_scatter(cnt_v, [d], ones)

    @pl.when(c == 0)
    def _():
        run(EPW0, pl.multiple_of(s * PAIR, 128))

    @pl.when(c == 1)
    def _():
        run(EPW1, pl.multiple_of(s * PAIR + EPW0, 128))

    pltpu.sync_copy(T_v, T_hbm.at[wid])
    pltpu.sync_copy(cnt_v, cnt_hbm.at[wid])


def _segment_parts(sd, r, p):
    return _sc_segsum_kernel()(sd, r, p)


def kernel(x_a, edge_index_a2a, edge_attr_a2a, g_W, g_b, f_W, f_b, cls_W, cls_b):
    node_out, c_vec = pl.pallas_call(
        _prep_body,
        out_shape=[jax.ShapeDtypeStruct((8, N_NODES), jnp.float32),
                   jax.ShapeDtypeStruct((EA, 1), jnp.float32)],
    )(x_a, g_W, g_b, f_W, f_b, cls_W, cls_b)

    eaT = edge_attr_a2a.T
    EB = EPAD // 10
    r2, sd2 = pl.pallas_call(
        _edge_body,
        grid=(10,),
        in_specs=[pl.BlockSpec((EA, EB), lambda i: (0, i)),
                  pl.BlockSpec((2, EB), lambda i: (0, i)),
                  pl.BlockSpec((EA, 1), lambda i: (0, 0))],
        out_specs=[pl.BlockSpec((1, EB), lambda i: (0, i)),
                   pl.BlockSpec((1, EB), lambda i: (0, i))],
        out_shape=[jax.ShapeDtypeStruct((1, EPAD), jnp.float32),
                   jax.ShapeDtypeStruct((1, EPAD), jnp.int32)],
    )(eaT, edge_index_a2a, c_vec)
    p = node_out[2]

    T_parts, cnt_parts = _segment_parts(sd2, r2, p)

    out_row = pl.pallas_call(
        _combine_body,
        out_shape=jax.ShapeDtypeStruct((1, N_NODES), jnp.float32),
    )(node_out, T_parts, cnt_parts)
    return out_row.reshape(N_NODES, 1)
